# baseline (speedup 1.0000x reference)
import dataclasses

import jax
import jax.numpy as jnp
from jax import lax
from jax.experimental import pallas as pl
from jax.experimental.pallas import tpu as pltpu
from jax.experimental.pallas import tpu_sc as plsc

_N = 10000
_E = 320000
_D = 128
_NC = 2
_NS = 16
_NW = _NC * _NS
_CH = 80
_EPW = _E // _NW
_NBLK = _EPW // _CH
_NPAD = 10240
_ZR = _NPAD // _NS


def _sc_compiler_params():
    cp = pltpu.CompilerParams()
    if "needs_layout_passes" in pltpu.CompilerParams.__dataclass_fields__:
        cp = dataclasses.replace(cp, needs_layout_passes=False)
    return cp


def _sc_edge_pass(src, dst, asrc, adst, mvec, h, zrows, zden):
    mesh = plsc.VectorSubcoreMesh(core_axis_name="c", subcore_axis_name="s")

    @pl.kernel(
        out_type=[
            jax.ShapeDtypeStruct((_NC, _N, _D), jnp.float32),
            jax.ShapeDtypeStruct((_NC, _NPAD), jnp.float32),
        ],
        mesh=mesh,
        scratch_types=[
            pltpu.VMEM((_N,), jnp.float32),
            pltpu.VMEM((_N,), jnp.float32),
            pltpu.VMEM((16,), jnp.float32),
            pltpu.VMEM((4, _CH), jnp.int32),
            pltpu.VMEM((4, _CH), jnp.int32),
            pltpu.VMEM((4, _CH), jnp.float32),
            pltpu.VMEM((2, _CH, _D), jnp.float32),
            pltpu.VMEM_SHARED((_NPAD, _D), jnp.float32),
            pltpu.VMEM_SHARED((_NPAD,), jnp.float32),
            pltpu.SemaphoreType.DMA,
            pltpu.SemaphoreType.DMA,
            pltpu.SemaphoreType.DMA,
            pltpu.SemaphoreType.DMA,
        ],
        compiler_params=_sc_compiler_params(),
    )
    def edge_kernel(src_r, dst_r, asrc_r, adst_r, m_r, h_r, zr_r, zd_r,
                    accp_r, denp_r,
                    asrc_t, adst_t, m_t, srcb, dstb, pb, rows,
                    acc_sp, den_sp, gsem0, gsem1, ssem0, ssem1):
        c = lax.axis_index("c")
        s = lax.axis_index("s")
        w = c * _NS + s
        base_w = w * _EPW
        gsem = (gsem0, gsem1)
        ssem = (ssem0, ssem1)

        pltpu.sync_copy(asrc_r, asrc_t)
        pltpu.sync_copy(adst_r, adst_t)
        pltpu.sync_copy(m_r, m_t)

        pltpu.sync_copy(zr_r, acc_sp.at[pl.ds(s * _ZR, _ZR)])
        pltpu.sync_copy(zd_r, den_sp.at[pl.ds(s * _ZR, _ZR)])
        plsc.subcore_barrier()

        mv = m_t[...]

        def idx_load(i, q):
            pltpu.sync_copy(src_r.at[pl.ds(base_w + i * _CH, _CH)],
                            srcb.at[q])
            pltpu.sync_copy(dst_r.at[pl.ds(base_w + i * _CH, _CH)],
                            dstb.at[q])

        def gather_start(i, q, r):
            pltpu.async_copy(h_r.at[srcb.at[q]], rows.at[r], gsem[r])

        def pb_compute(q):
            @pl.loop(0, _CH, step=16)
            def _group(j):
                si = srcb[q, pl.ds(j, 16)]
                di = dstb[q, pl.ds(j, 16)]
                a = plsc.load_gather(asrc_t, [si])
                b = plsc.load_gather(adst_t, [di])
                e = a + b
                e = jnp.where(e >= 0.0, e, e * 0.2)
                pb[q, pl.ds(j, 16)] = jnp.exp(e - mv)

        def scale(q, r):
            @pl.loop(0, _CH, step=4)
            def _scale(rr):
                for u in range(4):
                    pv = plsc.load_gather(
                        pb.at[q], [jnp.broadcast_to(rr + u, (16,))])
                    for k in range(0, _D, 16):
                        rows[r, rr + u, pl.ds(k, 16)] = \
                            rows[r, rr + u, pl.ds(k, 16)] * pv

        def scatter_start(q, r):
            pltpu.async_copy(rows.at[r], acc_sp.at[dstb.at[q]], ssem[r],
                             add=True)
            pltpu.sync_copy(pb.at[q], den_sp.at[dstb.at[q]], add=True)

        def drain_scatter(r):
            pltpu.make_async_copy(h_r.at[pl.ds(0, _CH)], rows.at[r],
                                  ssem[r]).wait()

        def step(i, o, first=False, last=False):
            q, r = o % 4, o % 2
            pltpu.make_async_copy(h_r.at[pl.ds(0, _CH)], rows.at[r],
                                  gsem[r]).wait()
            scale(q, r)
            scatter_start(q, r)
            if not last:
                idx_load(i + 2, (o + 2) % 4)
            if not first:
                drain_scatter((r + 1) % 2)
            if not last:
                gather_start(i + 1, (o + 1) % 4, (r + 1) % 2)
                pb_compute((o + 1) % 4)

        idx_load(0, 0)
        idx_load(1, 1)
        gather_start(0, 0, 0)
        pb_compute(0)
        step(0, 0, first=True)
        step(1, 1)
        step(2, 2)
        step(3, 3)

        @pl.loop(1, (_NBLK - 1) // 4, step=1)
        def _steady(j):
            i0 = j * 4
            step(i0, 0)
            step(i0 + 1, 1)
            step(i0 + 2, 2)
            step(i0 + 3, 3)

        step(_NBLK - 1, (_NBLK - 1) % 4, last=True)
        drain_scatter((_NBLK - 1) % 2)

        plsc.subcore_barrier()

        @pl.when(s == 0)
        def _writeback():
            pltpu.sync_copy(acc_sp.at[pl.ds(0, _N)], accp_r.at[c])
            pltpu.sync_copy(den_sp, denp_r.at[c])

    return edge_kernel(src, dst, asrc, adst, mvec, h, zrows, zden)


def _attn_tail(h, a_s, a_d, as_o, ad_o, m_o):
    asr = jnp.sum(h * a_s[None, :], axis=1)
    adr = jnp.sum(h * a_d[None, :], axis=1)
    as_o[...] = asr
    ad_o[...] = adr
    m = jnp.max(asr) + jnp.max(adr)
    m = jnp.where(m >= 0.0, m, m * 0.2)
    m_o[...] = jnp.broadcast_to(m, (16,))


_TC_OUT_TYPES = [
    jax.ShapeDtypeStruct((_N, _D), jnp.float32),
    jax.ShapeDtypeStruct((_N,), jnp.float32),
    jax.ShapeDtypeStruct((_N,), jnp.float32),
    jax.ShapeDtypeStruct((16,), jnp.float32),
]


def _tc_in(x, W1, a_s, a_d):
    def body(x_r, w_r, as_r, ad_r, h_o, as_o, ad_o, m_o):
        h = lax.dot_general(x_r[...], w_r[...], (((1,), (1,)), ((), ())),
                            preferred_element_type=jnp.float32)
        h_o[...] = h
        _attn_tail(h, as_r[...], ad_r[...], as_o, ad_o, m_o)

    return pl.pallas_call(body, out_shape=_TC_OUT_TYPES)(x, W1, a_s, a_d)


def _norm_relu(acc_r, den_r, b):
    den = den_r[0, : _N] + den_r[1, : _N] + 1e-16
    hs = acc_r[0] + acc_r[1]
    h = hs / den[:, None] + b[None, :]
    return jnp.maximum(h, 0.0)


def _tc_mid(accp, denp, b, W, a_s, a_d):
    def body(acc_r, den_r, b_r, w_r, as_r, ad_r, h_o, as_o, ad_o, m_o):
        h1 = _norm_relu(acc_r, den_r, b_r[...])
        h = lax.dot_general(h1, w_r[...], (((1,), (1,)), ((), ())),
                            preferred_element_type=jnp.float32)
        h_o[...] = h
        _attn_tail(h, as_r[...], ad_r[...], as_o, ad_o, m_o)

    return pl.pallas_call(body, out_shape=_TC_OUT_TYPES)(
        accp, denp, b, W, a_s, a_d)


def _tc_out(accp, denp, b, fsW, fsb):
    def body(acc_r, den_r, b_r, w_r, wb_r, o_r):
        h = _norm_relu(acc_r, den_r, b_r[...])
        o_r[...] = lax.dot_general(h, w_r[...], (((1,), (1,)), ((), ())),
                                   preferred_element_type=jnp.float32) \
            + wb_r[...][None, :]

    return pl.pallas_call(
        body,
        out_shape=[jax.ShapeDtypeStruct((_N, 16), jnp.float32)],
    )(accp, denp, b, fsW, fsb)[0]


def kernel(x, edge_index, W1, att_src1, att_dst1, b1,
           W2, att_src2, att_dst2, b2, fW, fb, sW, sb):
    zpad = jnp.zeros((_CH,), jnp.int32)
    src = jnp.concatenate([edge_index[0].astype(jnp.int32), zpad])
    dst = jnp.concatenate([edge_index[1].astype(jnp.int32), zpad])
    zrows = jnp.zeros((_ZR, _D), jnp.float32)
    zden = jnp.zeros((_ZR,), jnp.float32)

    h1, as1, ad1, m1 = _tc_in(x, W1, att_src1, att_dst1)
    acc1, den1 = _sc_edge_pass(src, dst, as1, ad1, m1, h1, zrows, zden)
    h2, as2, ad2, m2 = _tc_mid(acc1, den1, b1, W2, att_src2, att_dst2)
    acc2, den2 = _sc_edge_pass(src, dst, as2, ad2, m2, h2, zrows, zden)

    fsW = jnp.concatenate([fW, sW, jnp.zeros((6, _D), jnp.float32)], axis=0)
    fsb = jnp.concatenate([fb, sb, jnp.zeros((6,), jnp.float32)], axis=0)
    out = _tc_out(acc2, den2, b2, fsW, fsb)
    return out[:, :3], out[:, 3:10]
